# baseline (device time: 9867 ns/iter reference)
import jax
import jax.numpy as jnp
from jax import lax
from jax.experimental import pallas as pl
from jax.experimental.pallas import tpu as pltpu

N_DEV = 8
N_TOK = 256
D_IN = 128
D_OUT = 256
N_EXP = 16
EXP_PER_DEV = 2
ROWS = N_TOK // N_DEV


def kernel(x, router_W, route_idx, expert_W):
    def body(x_ref, rw_ref, idx_ref, ew_ref, out_ref,
             partial_ref, diag_ref, w_ref, acc_ref,
             send_sems, recv_sems, credit_sems):
        my = lax.axis_index("i")
        td = my ^ 6

        bar = pltpu.get_barrier_semaphore()
        pl.semaphore_signal(bar, inc=1)
        pl.semaphore_wait(bar, 1)

        for k in range(1, N_DEV):
            s = lax.rem(my - k + N_DEV, N_DEV)
            pl.semaphore_signal(
                credit_sems.at[k], inc=1, device_id=(s,),
                device_id_type=pl.DeviceIdType.MESH,
            )

        xf = x_ref[:, :]
        scores = jnp.dot(xf, rw_ref[:, :], preferred_element_type=jnp.float32)
        smax = jnp.max(scores, axis=1, keepdims=True)
        es = jnp.exp(scores - smax)
        eidx = lax.broadcasted_iota(jnp.int32, (N_TOK, N_EXP), 1)
        i0 = idx_ref[:, 0:1]
        i1 = idx_ref[:, 1:2]
        p0 = jnp.sum(jnp.where(eidx == i0, es, 0.0), axis=1, keepdims=True)
        p1 = jnp.sum(jnp.where(eidx == i1, es, 0.0), axis=1, keepdims=True)
        gs = p0 + p1
        for le in range(EXP_PER_DEV):
            eg = my * EXP_PER_DEV + le
            w_ref[:, le:le + 1] = (jnp.where(i0 == eg, p0, 0.0)
                                   + jnp.where(i1 == eg, p1, 0.0)) / gs

        drows = pl.ds(td * ROWS, ROWS)
        xd = x_ref[drows, :].astype(jnp.bfloat16)
        pd = jnp.zeros((ROWS, D_OUT), jnp.float32)
        for le in range(EXP_PER_DEV):
            yd = jnp.dot(xd, ew_ref[le].astype(jnp.bfloat16),
                         preferred_element_type=jnp.float32)
            pd = pd + w_ref[drows, le:le + 1] * yd
        diag_ref[:, :] = pd.astype(jnp.bfloat16)

        def make_rdma(k, src):
            t = lax.rem(my + k, N_DEV)
            return pltpu.make_async_remote_copy(
                src_ref=src,
                dst_ref=acc_ref.at[k],
                send_sem=send_sems.at[k],
                recv_sem=recv_sems.at[k],
                device_id=(t,),
                device_id_type=pl.DeviceIdType.MESH,
            )

        for k in range(1, N_DEV):
            t = lax.rem(my + k, N_DEV)

            @pl.when(t == td)
            def _(k=k):
                pl.semaphore_wait(credit_sems.at[k], 1)
                make_rdma(k, diag_ref).start()

        xb = xf.astype(jnp.bfloat16)
        partial = jnp.zeros((N_TOK, D_OUT), jnp.float32)
        for le in range(EXP_PER_DEV):
            y = jnp.dot(xb, ew_ref[le].astype(jnp.bfloat16),
                        preferred_element_type=jnp.float32)
            partial = partial + w_ref[:, le:le + 1] * y
        partial_ref[:, :] = partial.astype(jnp.bfloat16)

        waiters = []
        for k in range(1, N_DEV):
            t = lax.rem(my + k, N_DEV)
            rdma = make_rdma(k, partial_ref.at[pl.ds(t * ROWS, ROWS)])

            @pl.when(t != td)
            def _(k=k, rdma=rdma):
                pl.semaphore_wait(credit_sems.at[k], 1)
                rdma.start()

            waiters.append(rdma)

        out = partial_ref[pl.ds(my * ROWS, ROWS), :].astype(jnp.float32)
        for rdma in waiters:
            rdma.wait_recv()
        for k in range(1, N_DEV):
            out = out + acc_ref[k].astype(jnp.float32)
        out_ref[:, :] = out
        for rdma in waiters:
            rdma.wait_send()

    return pl.pallas_call(
        body,
        out_shape=jax.ShapeDtypeStruct((ROWS, D_OUT), jnp.float32),
        in_specs=[pl.BlockSpec(memory_space=pltpu.VMEM)] * 4,
        out_specs=pl.BlockSpec(memory_space=pltpu.VMEM),
        scratch_shapes=[
            pltpu.VMEM((N_TOK, D_OUT), jnp.bfloat16),
            pltpu.VMEM((ROWS, D_OUT), jnp.bfloat16),
            pltpu.VMEM((N_TOK, EXP_PER_DEV), jnp.float32),
            pltpu.VMEM((N_DEV, ROWS, D_OUT), jnp.bfloat16),
            pltpu.SemaphoreType.DMA((N_DEV,)),
            pltpu.SemaphoreType.DMA((N_DEV,)),
            pltpu.SemaphoreType.REGULAR((N_DEV,)),
        ],
        compiler_params=pltpu.CompilerParams(collective_id=0),
    )(x, router_W, route_idx, expert_W)


# device time: 9368 ns/iter; 1.0533x vs baseline; 1.0533x over previous
import jax
import jax.numpy as jnp
from jax import lax
from jax.experimental import pallas as pl
from jax.experimental.pallas import tpu as pltpu

N_DEV = 8
N_TOK = 256
D_IN = 128
D_OUT = 256
N_EXP = 16
EXP_PER_DEV = 2
ROWS = N_TOK // N_DEV


def kernel(x, router_W, route_idx, expert_W):
    def body(x_ref, rw_ref, idx_ref, ew_ref, out_ref,
             partial_ref, diag_ref, w_ref, acc_ref,
             send_sems, recv_sems):
        my = lax.axis_index("i")
        td = my ^ 6

        bar = pltpu.get_barrier_semaphore()
        pl.semaphore_signal(bar, inc=1)
        pl.semaphore_wait(bar, 1)

        xf = x_ref[:, :]
        scores = jnp.dot(xf, rw_ref[:, :], preferred_element_type=jnp.float32)
        smax = jnp.max(scores, axis=1, keepdims=True)
        es = jnp.exp(scores - smax)
        eidx = lax.broadcasted_iota(jnp.int32, (N_TOK, N_EXP), 1)
        i0 = idx_ref[:, 0:1]
        i1 = idx_ref[:, 1:2]
        p0 = jnp.sum(jnp.where(eidx == i0, es, 0.0), axis=1, keepdims=True)
        p1 = jnp.sum(jnp.where(eidx == i1, es, 0.0), axis=1, keepdims=True)
        gs = p0 + p1
        for le in range(EXP_PER_DEV):
            eg = my * EXP_PER_DEV + le
            w_ref[:, le:le + 1] = (jnp.where(i0 == eg, p0, 0.0)
                                   + jnp.where(i1 == eg, p1, 0.0)) / gs

        drows = pl.ds(td * ROWS, ROWS)
        xd = x_ref[drows, :].astype(jnp.bfloat16)
        pd = jnp.zeros((ROWS, D_OUT), jnp.float32)
        for le in range(EXP_PER_DEV):
            yd = jnp.dot(xd, ew_ref[le].astype(jnp.bfloat16),
                         preferred_element_type=jnp.float32)
            pd = pd + w_ref[drows, le:le + 1] * yd
        diag_ref[:, :] = pd.astype(jnp.bfloat16)

        def make_rdma(k, src):
            t = lax.rem(my + k, N_DEV)
            return pltpu.make_async_remote_copy(
                src_ref=src,
                dst_ref=acc_ref.at[k],
                send_sem=send_sems.at[k],
                recv_sem=recv_sems.at[k],
                device_id=(t,),
                device_id_type=pl.DeviceIdType.MESH,
            )

        for k in range(1, N_DEV):
            t = lax.rem(my + k, N_DEV)

            @pl.when(t == td)
            def _(k=k):
                make_rdma(k, diag_ref).start()

        xb = xf.astype(jnp.bfloat16)
        partial = jnp.zeros((N_TOK, D_OUT), jnp.float32)
        for le in range(EXP_PER_DEV):
            y = jnp.dot(xb, ew_ref[le].astype(jnp.bfloat16),
                        preferred_element_type=jnp.float32)
            partial = partial + w_ref[:, le:le + 1] * y
        partial_ref[:, :] = partial.astype(jnp.bfloat16)

        waiters = []
        for k in range(1, N_DEV):
            t = lax.rem(my + k, N_DEV)
            rdma = make_rdma(k, partial_ref.at[pl.ds(t * ROWS, ROWS)])

            @pl.when(t != td)
            def _(k=k, rdma=rdma):
                rdma.start()

            waiters.append(rdma)

        out = partial_ref[pl.ds(my * ROWS, ROWS), :].astype(jnp.float32)
        for rdma in waiters:
            rdma.wait_recv()
        for k in range(1, N_DEV):
            out = out + acc_ref[k].astype(jnp.float32)
        out_ref[:, :] = out
        for rdma in waiters:
            rdma.wait_send()

    return pl.pallas_call(
        body,
        out_shape=jax.ShapeDtypeStruct((ROWS, D_OUT), jnp.float32),
        in_specs=[pl.BlockSpec(memory_space=pltpu.VMEM)] * 4,
        out_specs=pl.BlockSpec(memory_space=pltpu.VMEM),
        scratch_shapes=[
            pltpu.VMEM((N_TOK, D_OUT), jnp.bfloat16),
            pltpu.VMEM((ROWS, D_OUT), jnp.bfloat16),
            pltpu.VMEM((N_TOK, EXP_PER_DEV), jnp.float32),
            pltpu.VMEM((N_DEV, ROWS, D_OUT), jnp.bfloat16),
            pltpu.SemaphoreType.DMA((N_DEV,)),
            pltpu.SemaphoreType.DMA((N_DEV,)),
        ],
        compiler_params=pltpu.CompilerParams(collective_id=0),
    )(x, router_W, route_idx, expert_W)


# device time: 9282 ns/iter; 1.0630x vs baseline; 1.0093x over previous
import jax
import jax.numpy as jnp
from jax import lax
from jax.experimental import pallas as pl
from jax.experimental.pallas import tpu as pltpu

N_DEV = 8
N_TOK = 256
D_IN = 128
D_OUT = 256
N_EXP = 16
EXP_PER_DEV = 2
ROWS = N_TOK // N_DEV


def kernel(x, router_W, route_idx, expert_W):
    def body(x_ref, rw_ref, idx_ref, ew_ref, out_ref,
             partial_ref, diag_ref, acc_ref,
             send_sems, recv_sems):
        my = lax.axis_index("i")
        td = my ^ 6

        def gate_weights(xs, ii0, ii1, n):
            scores = jnp.dot(xs, rw_ref[:, :],
                             preferred_element_type=jnp.float32)
            smax = jnp.max(scores, axis=1, keepdims=True)
            es = jnp.exp(scores - smax)
            eidx = lax.broadcasted_iota(jnp.int32, (n, N_EXP), 1)
            p0 = jnp.sum(jnp.where(eidx == ii0, es, 0.0), axis=1,
                         keepdims=True)
            p1 = jnp.sum(jnp.where(eidx == ii1, es, 0.0), axis=1,
                         keepdims=True)
            gs = p0 + p1
            ws = []
            for le in range(EXP_PER_DEV):
                eg = my * EXP_PER_DEV + le
                ws.append((jnp.where(ii0 == eg, p0, 0.0)
                           + jnp.where(ii1 == eg, p1, 0.0)) / gs)
            return ws

        drows = pl.ds(td * ROWS, ROWS)
        xdf = x_ref[drows, :]
        wd = gate_weights(xdf, idx_ref[drows, 0:1], idx_ref[drows, 1:2], ROWS)
        xd = xdf.astype(jnp.bfloat16)
        pd = jnp.zeros((ROWS, D_OUT), jnp.float32)
        for le in range(EXP_PER_DEV):
            yd = jnp.dot(xd, ew_ref[le].astype(jnp.bfloat16),
                         preferred_element_type=jnp.float32)
            pd = pd + wd[le] * yd
        diag_ref[:, :] = pd.astype(jnp.bfloat16)

        def make_rdma(k, src):
            t = lax.rem(my + k, N_DEV)
            return pltpu.make_async_remote_copy(
                src_ref=src,
                dst_ref=acc_ref.at[k],
                send_sem=send_sems.at[k],
                recv_sem=recv_sems.at[k],
                device_id=(t,),
                device_id_type=pl.DeviceIdType.MESH,
            )

        for k in range(1, N_DEV):
            t = lax.rem(my + k, N_DEV)

            @pl.when(t == td)
            def _(k=k):
                make_rdma(k, diag_ref).start()

        xf = x_ref[:, :]
        wf = gate_weights(xf, idx_ref[:, 0:1], idx_ref[:, 1:2], N_TOK)
        xb = xf.astype(jnp.bfloat16)
        partial = jnp.zeros((N_TOK, D_OUT), jnp.float32)
        for le in range(EXP_PER_DEV):
            y = jnp.dot(xb, ew_ref[le].astype(jnp.bfloat16),
                        preferred_element_type=jnp.float32)
            partial = partial + wf[le] * y
        partial_ref[:, :] = partial.astype(jnp.bfloat16)

        waiters = []
        for k in range(1, N_DEV):
            t = lax.rem(my + k, N_DEV)
            rdma = make_rdma(k, partial_ref.at[pl.ds(t * ROWS, ROWS)])

            @pl.when(t != td)
            def _(k=k, rdma=rdma):
                rdma.start()

            waiters.append(rdma)

        out = partial_ref[pl.ds(my * ROWS, ROWS), :].astype(jnp.float32)
        for rdma in waiters:
            rdma.wait_recv()
        for k in range(1, N_DEV):
            out = out + acc_ref[k].astype(jnp.float32)
        out_ref[:, :] = out
        for rdma in waiters:
            rdma.wait_send()

        bar = pltpu.get_barrier_semaphore()
        pl.semaphore_signal(bar, inc=1)
        pl.semaphore_wait(bar, 1)

    return pl.pallas_call(
        body,
        out_shape=jax.ShapeDtypeStruct((ROWS, D_OUT), jnp.float32),
        in_specs=[pl.BlockSpec(memory_space=pltpu.VMEM)] * 4,
        out_specs=pl.BlockSpec(memory_space=pltpu.VMEM),
        scratch_shapes=[
            pltpu.VMEM((N_TOK, D_OUT), jnp.bfloat16),
            pltpu.VMEM((ROWS, D_OUT), jnp.bfloat16),
            pltpu.VMEM((N_DEV, ROWS, D_OUT), jnp.bfloat16),
            pltpu.SemaphoreType.DMA((N_DEV,)),
            pltpu.SemaphoreType.DMA((N_DEV,)),
        ],
        compiler_params=pltpu.CompilerParams(collective_id=0),
    )(x, router_W, route_idx, expert_W)
